# baseline (device time: 49988 ns/iter reference)
import functools

import jax
import jax.numpy as jnp
from jax import lax
from jax.experimental import pallas as pl
from jax.experimental.pallas import tpu as pltpu

N_DEV = 4
SQ = 1024
SKV = 1024
HQ = 8
DH = 128
D = HQ * DH
BLK = 64
SCALE = 0.08838834764831843

NC = 8
CR = SQ // NC


def kernel(x, Wq, K_ext, V_ext, Wo):
    def body(x_ref, wq_ref, k_ref, v_ref, wo_ref, out_ref,
             kvflat, cbuf, sem1, sem3, recv_sems):
        my = lax.axis_index("i")
        left = (my - 1) % N_DEV
        right = (my + 1) % N_DEV

        barrier_sem = pltpu.get_barrier_semaphore()
        for nbr in (left, right):
            pl.semaphore_signal(
                barrier_sem, inc=1,
                device_id=(nbr,), device_id_type=pl.DeviceIdType.MESH,
            )
        pl.semaphore_wait(barrier_sem, 2)

        def chunk_ref(c):
            return cbuf.at[pl.ds(c * CR, CR), :]

        def project_chunk(c):
            rows = pl.ds(c * CR, CR)
            ctxc = cbuf[rows, :].astype(jnp.float32)
            out_ref[0, rows, :] = jnp.dot(
                ctxc, wo_ref[...], preferred_element_type=jnp.float32)

        @pl.when(my == 0)
        def _():
            kvflat[0] = k_ref[0].reshape(SKV, D).astype(jnp.bfloat16)
            kvflat[1] = v_ref[0].reshape(SKV, D).astype(jnp.bfloat16)
            kbuf = kvflat[0]
            vbuf = kvflat[1]
            q = (jnp.dot(x_ref[0], wq_ref[...],
                         preferred_element_type=jnp.float32)
                 * (SCALE * 1.4426950408889634)).astype(jnp.bfloat16)
            dmask = (
                lax.broadcasted_iota(jnp.int32, (CR, CR), 0) // BLK
                >= lax.broadcasted_iota(jnp.int32, (CR, CR), 1) // BLK
            )
            rdmas = []
            for c in range(NC):
                rows = pl.ds(c * CR, CR)
                LL = c * CR
                ctx_parts = []
                for h in range(HQ):
                    qh = q[c * CR:(c + 1) * CR, h * DH:(h + 1) * DH]
                    s_d = lax.dot_general(
                        qh, kbuf[LL:LL + CR, h * DH:(h + 1) * DH],
                        (((1,), (1,)), ((), ())),
                        preferred_element_type=jnp.float32,
                    )
                    e_d = jnp.where(dmask, jnp.exp2(s_d), 0.0)
                    den = jnp.sum(e_d, axis=1, keepdims=True)
                    ctx_u = jnp.dot(e_d.astype(jnp.bfloat16),
                                    vbuf[LL:LL + CR, h * DH:(h + 1) * DH],
                                    preferred_element_type=jnp.float32)
                    if c > 0:
                        s_l = lax.dot_general(
                            qh, kbuf[0:LL, h * DH:(h + 1) * DH],
                            (((1,), (1,)), ((), ())),
                            preferred_element_type=jnp.float32,
                        )
                        e_l = jnp.exp2(s_l)
                        den = den + jnp.sum(e_l, axis=1, keepdims=True)
                        ctx_u = ctx_u + jnp.dot(
                            e_l.astype(jnp.bfloat16),
                            vbuf[0:LL, h * DH:(h + 1) * DH],
                            preferred_element_type=jnp.float32)
                    ctx_parts.append(ctx_u * (1.0 / den))
                ctx = jnp.concatenate(ctx_parts, axis=-1)
                cbuf[rows, :] = ctx.astype(jnp.bfloat16)
                for tgt, sems in ((1, sem1), (3, sem3)):
                    r = pltpu.make_async_remote_copy(
                        src_ref=chunk_ref(c), dst_ref=chunk_ref(c),
                        send_sem=sems.at[c], recv_sem=recv_sems.at[c],
                        device_id=(tgt,),
                        device_id_type=pl.DeviceIdType.MESH,
                    )
                    r.start()
                    rdmas.append(r)
            for c in range(NC):
                project_chunk(c)
            for r in rdmas:
                r.wait_send()

        @pl.when(my == 1)
        def _():
            fwds = []
            for c in range(NC):
                rc = pltpu.make_async_remote_copy(
                    src_ref=chunk_ref(c), dst_ref=chunk_ref(c),
                    send_sem=sem1.at[c], recv_sem=recv_sems.at[c],
                    device_id=(2,), device_id_type=pl.DeviceIdType.MESH,
                )
                rc.wait_recv()
                rc.start()
                fwds.append(rc)
                project_chunk(c)
            for r in fwds:
                r.wait_send()

        @pl.when((my == 2) | (my == 3))
        def _():
            for c in range(NC):
                rc = pltpu.make_async_remote_copy(
                    src_ref=chunk_ref(c), dst_ref=chunk_ref(c),
                    send_sem=sem1.at[c], recv_sem=recv_sems.at[c],
                    device_id=(0,), device_id_type=pl.DeviceIdType.MESH,
                )
                rc.wait_recv()
                project_chunk(c)

        @functools.partial(pl.run_scoped,
                           second_barrier=pltpu.SemaphoreType.REGULAR)
        def _(second_barrier):
            for nbr in (left, right):
                pl.semaphore_signal(
                    second_barrier, inc=1,
                    device_id=(nbr,), device_id_type=pl.DeviceIdType.MESH,
                )
            pl.semaphore_wait(second_barrier, 2)

    return pl.pallas_call(
        body,
        out_shape=jax.ShapeDtypeStruct((1, SQ, D), jnp.float32),
        in_specs=[pl.BlockSpec(memory_space=pltpu.VMEM)] * 5,
        out_specs=pl.BlockSpec(memory_space=pltpu.VMEM),
        scratch_shapes=[
            pltpu.VMEM((2, SKV, D), jnp.bfloat16),
            pltpu.VMEM((SQ, D), jnp.bfloat16),
            pltpu.SemaphoreType.DMA((NC,)),
            pltpu.SemaphoreType.DMA((NC,)),
            pltpu.SemaphoreType.DMA((NC,)),
        ],
        compiler_params=pltpu.CompilerParams(collective_id=0),
    )(x, Wq, K_ext, V_ext, Wo)


# device time: 48141 ns/iter; 1.0384x vs baseline; 1.0384x over previous
import functools

import jax
import jax.numpy as jnp
from jax import lax
from jax.experimental import pallas as pl
from jax.experimental.pallas import tpu as pltpu

N_DEV = 4
SQ = 1024
SKV = 1024
HQ = 8
DH = 128
D = HQ * DH
BLK = 64
SCALE = 0.08838834764831843

NC = 8
CR = SQ // NC


def kernel(x, Wq, K_ext, V_ext, Wo):
    def body(x_ref, wq_ref, k_ref, v_ref, wo_ref, out_ref,
             kvflat, cbuf, wob, sem1, sem3, semd, recv_sems):
        my = lax.axis_index("i")
        left = (my - 1) % N_DEV
        right = (my + 1) % N_DEV

        barrier_sem = pltpu.get_barrier_semaphore()
        for nbr in (left, right, (my + 2) % N_DEV):
            pl.semaphore_signal(
                barrier_sem, inc=1,
                device_id=(nbr,), device_id_type=pl.DeviceIdType.MESH,
            )
        pl.semaphore_wait(barrier_sem, 3)

        def chunk_ref(c):
            return cbuf.at[pl.ds(c * CR, CR), :]

        def project_chunk(c):
            rows = pl.ds(c * CR, CR)
            out_ref[0, rows, :] = jnp.dot(
                cbuf[rows, :], wob[...],
                preferred_element_type=jnp.float32)

        @pl.when(my == 0)
        def _():
            kvflat[0] = k_ref[0].astype(jnp.bfloat16).reshape(SKV, D)
            kvflat[1] = v_ref[0].astype(jnp.bfloat16).reshape(SKV, D)
            kbuf = kvflat[0]
            vbuf = kvflat[1]
            q = (jnp.dot(x_ref[0].astype(jnp.bfloat16),
                         wq_ref[...].astype(jnp.bfloat16),
                         preferred_element_type=jnp.float32)
                 * (SCALE * 1.4426950408889634)).astype(jnp.bfloat16)
            dmask = (
                lax.broadcasted_iota(jnp.int32, (CR, CR), 0) // BLK
                >= lax.broadcasted_iota(jnp.int32, (CR, CR), 1) // BLK
            )
            rdmas = []
            for c in range(NC):
                rows = pl.ds(c * CR, CR)
                LL = c * CR
                ctx_parts = []
                for h in range(HQ):
                    qh = q[c * CR:(c + 1) * CR, h * DH:(h + 1) * DH]
                    s_d = lax.dot_general(
                        qh, kbuf[LL:LL + CR, h * DH:(h + 1) * DH],
                        (((1,), (1,)), ((), ())),
                        preferred_element_type=jnp.float32,
                    )
                    e_d = jnp.where(dmask, jnp.exp2(s_d), 0.0)
                    den = jnp.sum(e_d, axis=1, keepdims=True)
                    ctx_u = jnp.dot(e_d.astype(jnp.bfloat16),
                                    vbuf[LL:LL + CR, h * DH:(h + 1) * DH],
                                    preferred_element_type=jnp.float32)
                    if c > 0:
                        s_l = lax.dot_general(
                            qh, kbuf[0:LL, h * DH:(h + 1) * DH],
                            (((1,), (1,)), ((), ())),
                            preferred_element_type=jnp.float32,
                        )
                        e_l = jnp.exp2(s_l)
                        den = den + jnp.sum(e_l, axis=1, keepdims=True)
                        ctx_u = ctx_u + jnp.dot(
                            e_l.astype(jnp.bfloat16),
                            vbuf[0:LL, h * DH:(h + 1) * DH],
                            preferred_element_type=jnp.float32)
                    ctx_parts.append(ctx_u * (1.0 / den))
                ctx = jnp.concatenate(ctx_parts, axis=-1)
                cbuf[rows, :] = ctx.astype(jnp.bfloat16)
                targets = [(1, sem1.at[c]), (3, sem3.at[c])]
                if c == NC - 1:
                    targets.append((2, semd.at[0]))
                for tgt, ssem in targets:
                    r = pltpu.make_async_remote_copy(
                        src_ref=chunk_ref(c), dst_ref=chunk_ref(c),
                        send_sem=ssem, recv_sem=recv_sems.at[c],
                        device_id=(tgt,),
                        device_id_type=pl.DeviceIdType.MESH,
                    )
                    r.start()
                    rdmas.append(r)
            wob[...] = wo_ref[...].astype(jnp.bfloat16)
            for c in range(NC):
                project_chunk(c)
            for r in rdmas:
                r.wait_send()

        @pl.when(my == 1)
        def _():
            wob[...] = wo_ref[...].astype(jnp.bfloat16)
            fwds = []
            for c in range(NC):
                rc = pltpu.make_async_remote_copy(
                    src_ref=chunk_ref(c), dst_ref=chunk_ref(c),
                    send_sem=sem1.at[c], recv_sem=recv_sems.at[c],
                    device_id=(2,), device_id_type=pl.DeviceIdType.MESH,
                )
                rc.wait_recv()
                if c < NC - 1:
                    rc.start()
                    fwds.append(rc)
                project_chunk(c)
            for r in fwds:
                r.wait_send()

        @pl.when((my == 2) | (my == 3))
        def _():
            wob[...] = wo_ref[...].astype(jnp.bfloat16)
            for c in range(NC):
                rc = pltpu.make_async_remote_copy(
                    src_ref=chunk_ref(c), dst_ref=chunk_ref(c),
                    send_sem=sem1.at[c], recv_sem=recv_sems.at[c],
                    device_id=(0,), device_id_type=pl.DeviceIdType.MESH,
                )
                rc.wait_recv()
                project_chunk(c)

        @functools.partial(pl.run_scoped,
                           second_barrier=pltpu.SemaphoreType.REGULAR)
        def _(second_barrier):
            for nbr in (left, right):
                pl.semaphore_signal(
                    second_barrier, inc=1,
                    device_id=(nbr,), device_id_type=pl.DeviceIdType.MESH,
                )
            pl.semaphore_wait(second_barrier, 2)

    return pl.pallas_call(
        body,
        out_shape=jax.ShapeDtypeStruct((1, SQ, D), jnp.float32),
        in_specs=[pl.BlockSpec(memory_space=pltpu.VMEM)] * 5,
        out_specs=pl.BlockSpec(memory_space=pltpu.VMEM),
        scratch_shapes=[
            pltpu.VMEM((2, SKV, D), jnp.bfloat16),
            pltpu.VMEM((SQ, D), jnp.bfloat16),
            pltpu.VMEM((D, D), jnp.bfloat16),
            pltpu.SemaphoreType.DMA((NC,)),
            pltpu.SemaphoreType.DMA((NC,)),
            pltpu.SemaphoreType.DMA((1,)),
            pltpu.SemaphoreType.DMA((NC,)),
        ],
        compiler_params=pltpu.CompilerParams(collective_id=0),
    )(x, Wq, K_ext, V_ext, Wo)


# device time: 46432 ns/iter; 1.0766x vs baseline; 1.0368x over previous
import functools

import jax
import jax.numpy as jnp
from jax import lax
from jax.experimental import pallas as pl
from jax.experimental.pallas import tpu as pltpu

N_DEV = 4
SQ = 1024
SKV = 1024
HQ = 8
DH = 128
D = HQ * DH
BLK = 64
SCALE = 0.08838834764831843
LOG2E = 1.4426950408889634

NC = 8
CR = SQ // NC


def kernel(x, Wq, K_ext, V_ext, Wo):
    def body(x_ref, wq_ref, k_ref, v_ref, wo_ref, out_ref,
             xv, wqv, kraw, vraw, wov,
             kvflat, cbuf, wob,
             dsems, sem1, sem3, semd, recv_sems):
        my = lax.axis_index("i")
        left = (my - 1) % N_DEV
        right = (my + 1) % N_DEV

        dma_wo = pltpu.make_async_copy(wo_ref, wov, dsems.at[4])
        dma_wo.start()
        dma_x = pltpu.make_async_copy(x_ref.at[0], xv, dsems.at[0])
        dma_wq = pltpu.make_async_copy(wq_ref, wqv, dsems.at[1])
        dma_k = pltpu.make_async_copy(k_ref.at[0], kraw, dsems.at[2])
        dma_v = pltpu.make_async_copy(v_ref.at[0], vraw, dsems.at[3])

        @pl.when(my == 0)
        def _():
            dma_x.start()
            dma_wq.start()
            dma_k.start()
            dma_v.start()

        barrier_sem = pltpu.get_barrier_semaphore()
        for nbr in (left, right, (my + 2) % N_DEV):
            pl.semaphore_signal(
                barrier_sem, inc=1,
                device_id=(nbr,), device_id_type=pl.DeviceIdType.MESH,
            )
        pl.semaphore_wait(barrier_sem, 3)

        def chunk_ref(c):
            return cbuf.at[pl.ds(c * CR, CR), :]

        def fill_wob():
            dma_wo.wait()
            wob[...] = wov[...].astype(jnp.bfloat16)

        def project_chunk(c):
            rows = pl.ds(c * CR, CR)
            out_ref[0, rows, :] = jnp.dot(
                cbuf[rows, :], wob[...],
                preferred_element_type=jnp.float32)

        @pl.when(my == 0)
        def _():
            dma_x.wait()
            dma_wq.wait()
            q = (jnp.dot(xv[...].astype(jnp.bfloat16),
                         wqv[...].astype(jnp.bfloat16),
                         preferred_element_type=jnp.float32)
                 * (SCALE * LOG2E)).astype(jnp.bfloat16)
            dma_k.wait()
            kvflat[0] = kraw[...].astype(jnp.bfloat16).reshape(SKV, D)
            dma_v.wait()
            kvflat[1] = vraw[...].astype(jnp.bfloat16).reshape(SKV, D)
            kbuf = kvflat[0]
            vbuf = kvflat[1]
            dmask = (
                lax.broadcasted_iota(jnp.int32, (CR, CR), 0) // BLK
                >= lax.broadcasted_iota(jnp.int32, (CR, CR), 1) // BLK
            )
            rdmas = []
            for c in range(NC):
                rows = pl.ds(c * CR, CR)
                LL = c * CR
                ctx_parts = []
                for h in range(HQ):
                    qh = q[c * CR:(c + 1) * CR, h * DH:(h + 1) * DH]
                    s_d = lax.dot_general(
                        qh, kbuf[LL:LL + CR, h * DH:(h + 1) * DH],
                        (((1,), (1,)), ((), ())),
                        preferred_element_type=jnp.float32,
                    )
                    e_d = jnp.where(dmask, jnp.exp2(s_d), 0.0)
                    den = jnp.sum(e_d, axis=1, keepdims=True)
                    ctx_u = jnp.dot(e_d.astype(jnp.bfloat16),
                                    vbuf[LL:LL + CR, h * DH:(h + 1) * DH],
                                    preferred_element_type=jnp.float32)
                    if c > 0:
                        s_l = lax.dot_general(
                            qh, kbuf[0:LL, h * DH:(h + 1) * DH],
                            (((1,), (1,)), ((), ())),
                            preferred_element_type=jnp.float32,
                        )
                        e_l = jnp.exp2(s_l)
                        den = den + jnp.sum(e_l, axis=1, keepdims=True)
                        ctx_u = ctx_u + jnp.dot(
                            e_l.astype(jnp.bfloat16),
                            vbuf[0:LL, h * DH:(h + 1) * DH],
                            preferred_element_type=jnp.float32)
                    ctx_parts.append(ctx_u * (1.0 / den))
                ctx = jnp.concatenate(ctx_parts, axis=-1)
                cbuf[rows, :] = ctx.astype(jnp.bfloat16)
                targets = [(1, sem1.at[c]), (3, sem3.at[c])]
                if c == NC - 1:
                    targets.append((2, semd.at[0]))
                for tgt, ssem in targets:
                    r = pltpu.make_async_remote_copy(
                        src_ref=chunk_ref(c), dst_ref=chunk_ref(c),
                        send_sem=ssem, recv_sem=recv_sems.at[c],
                        device_id=(tgt,),
                        device_id_type=pl.DeviceIdType.MESH,
                    )
                    r.start()
                    rdmas.append(r)
            fill_wob()
            for c in range(NC):
                project_chunk(c)
            for r in rdmas:
                r.wait_send()

        @pl.when(my == 1)
        def _():
            fill_wob()
            fwds = []
            for c in range(NC):
                rc = pltpu.make_async_remote_copy(
                    src_ref=chunk_ref(c), dst_ref=chunk_ref(c),
                    send_sem=sem1.at[c], recv_sem=recv_sems.at[c],
                    device_id=(2,), device_id_type=pl.DeviceIdType.MESH,
                )
                rc.wait_recv()
                if c < NC - 1:
                    rc.start()
                    fwds.append(rc)
                project_chunk(c)
            for r in fwds:
                r.wait_send()

        @pl.when((my == 2) | (my == 3))
        def _():
            fill_wob()
            for c in range(NC):
                rc = pltpu.make_async_remote_copy(
                    src_ref=chunk_ref(c), dst_ref=chunk_ref(c),
                    send_sem=sem1.at[c], recv_sem=recv_sems.at[c],
                    device_id=(0,), device_id_type=pl.DeviceIdType.MESH,
                )
                rc.wait_recv()
                project_chunk(c)


        @functools.partial(pl.run_scoped,
                           second_barrier=pltpu.SemaphoreType.REGULAR)
        def _(second_barrier):
            for nbr in (left, right):
                pl.semaphore_signal(
                    second_barrier, inc=1,
                    device_id=(nbr,), device_id_type=pl.DeviceIdType.MESH,
                )
            pl.semaphore_wait(second_barrier, 2)

    return pl.pallas_call(
        body,
        out_shape=jax.ShapeDtypeStruct((1, SQ, D), jnp.float32),
        in_specs=[pl.BlockSpec(memory_space=pl.ANY)] * 5,
        out_specs=pl.BlockSpec(memory_space=pltpu.VMEM),
        scratch_shapes=[
            pltpu.VMEM((SQ, D), jnp.float32),
            pltpu.VMEM((D, D), jnp.float32),
            pltpu.VMEM((SKV, HQ, DH), jnp.float32),
            pltpu.VMEM((SKV, HQ, DH), jnp.float32),
            pltpu.VMEM((D, D), jnp.float32),
            pltpu.VMEM((2, SKV, D), jnp.bfloat16),
            pltpu.VMEM((SQ, D), jnp.bfloat16),
            pltpu.VMEM((D, D), jnp.bfloat16),
            pltpu.SemaphoreType.DMA((5,)),
            pltpu.SemaphoreType.DMA((NC,)),
            pltpu.SemaphoreType.DMA((NC,)),
            pltpu.SemaphoreType.DMA((1,)),
            pltpu.SemaphoreType.DMA((NC,)),
        ],
        compiler_params=pltpu.CompilerParams(collective_id=0),
    )(x, Wq, K_ext, V_ext, Wo)


# device time: 43817 ns/iter; 1.1408x vs baseline; 1.0597x over previous
import functools

import jax
import jax.numpy as jnp
from jax import lax
from jax.experimental import pallas as pl
from jax.experimental.pallas import tpu as pltpu

N_DEV = 4
SQ = 1024
SKV = 1024
HQ = 8
DH = 128
D = HQ * DH
BLK = 64
SCALE = 0.08838834764831843
LOG2E = 1.4426950408889634

NC = 8
CR = SQ // NC


def kernel(x, Wq, K_ext, V_ext, Wo):
    def body(x_ref, wq_ref, k_ref, v_ref, wo_ref, out_ref,
             xv, wqv, kraw, vraw, wov,
             kvflat, cbuf, wob, obuf,
             dsems, kdsem, vdsem, odsem,
             sem1, sem3, semd, recv_sems):
        my = lax.axis_index("i")
        left = (my - 1) % N_DEV
        right = (my + 1) % N_DEV

        dma_wo = pltpu.make_async_copy(wo_ref, wov, dsems.at[2])
        dma_wo.start()
        dma_x = pltpu.make_async_copy(x_ref.at[0], xv, dsems.at[0])
        dma_wq = pltpu.make_async_copy(wq_ref, wqv, dsems.at[1])

        def kv_slab_dmas(c):
            rows = pl.ds(c * CR, CR)
            return (
                pltpu.make_async_copy(
                    k_ref.at[0, rows], kraw.at[rows], kdsem.at[c]),
                pltpu.make_async_copy(
                    v_ref.at[0, rows], vraw.at[rows], vdsem.at[c]),
            )

        @pl.when(my == 0)
        def _():
            dma_x.start()
            dma_wq.start()
            for c in range(NC):
                dk, dv = kv_slab_dmas(c)
                dk.start()
                dv.start()

        barrier_sem = pltpu.get_barrier_semaphore()
        for nbr in (left, right, (my + 2) % N_DEV):
            pl.semaphore_signal(
                barrier_sem, inc=1,
                device_id=(nbr,), device_id_type=pl.DeviceIdType.MESH,
            )
        pl.semaphore_wait(barrier_sem, 3)

        def chunk_ref(c):
            return cbuf.at[pl.ds(c * CR, CR), :]

        def fill_wob():
            dma_wo.wait()
            wob[...] = wov[...].astype(jnp.bfloat16)

        def project_chunk(c):
            rows = pl.ds(c * CR, CR)
            obuf[rows, :] = jnp.dot(
                cbuf[rows, :], wob[...],
                preferred_element_type=jnp.float32)
            od = pltpu.make_async_copy(
                obuf.at[rows], out_ref.at[0, rows], odsem.at[c])
            od.start()
            return od

        @pl.when(my == 0)
        def _():
            dma_x.wait()
            dma_wq.wait()
            q = (jnp.dot(xv[...].astype(jnp.bfloat16),
                         wqv[...].astype(jnp.bfloat16),
                         preferred_element_type=jnp.float32)
                 * (SCALE * LOG2E)).astype(jnp.bfloat16)
            dmask = (
                lax.broadcasted_iota(jnp.int32, (CR, CR), 0) // BLK
                >= lax.broadcasted_iota(jnp.int32, (CR, CR), 1) // BLK
            )
            kbuf = kvflat.at[0]
            vbuf = kvflat.at[1]
            rdmas = []
            for c in range(NC):
                rows = pl.ds(c * CR, CR)
                dk, dv = kv_slab_dmas(c)
                dk.wait()
                kvflat[0, c * CR:(c + 1) * CR, :] = (
                    kraw[c * CR:(c + 1) * CR]
                    .astype(jnp.bfloat16).reshape(CR, D))
                dv.wait()
                kvflat[1, c * CR:(c + 1) * CR, :] = (
                    vraw[c * CR:(c + 1) * CR]
                    .astype(jnp.bfloat16).reshape(CR, D))
                LL = c * CR
                ctx_parts = []
                for h in range(HQ):
                    qh = q[c * CR:(c + 1) * CR, h * DH:(h + 1) * DH]
                    s_d = lax.dot_general(
                        qh, kbuf[LL:LL + CR, h * DH:(h + 1) * DH],
                        (((1,), (1,)), ((), ())),
                        preferred_element_type=jnp.float32,
                    )
                    e_d = jnp.where(dmask, jnp.exp2(s_d), 0.0)
                    den = jnp.sum(e_d, axis=1, keepdims=True)
                    ctx_u = jnp.dot(e_d.astype(jnp.bfloat16),
                                    vbuf[LL:LL + CR, h * DH:(h + 1) * DH],
                                    preferred_element_type=jnp.float32)
                    if c > 0:
                        s_l = lax.dot_general(
                            qh, kbuf[0:LL, h * DH:(h + 1) * DH],
                            (((1,), (1,)), ((), ())),
                            preferred_element_type=jnp.float32,
                        )
                        e_l = jnp.exp2(s_l)
                        den = den + jnp.sum(e_l, axis=1, keepdims=True)
                        ctx_u = ctx_u + jnp.dot(
                            e_l.astype(jnp.bfloat16),
                            vbuf[0:LL, h * DH:(h + 1) * DH],
                            preferred_element_type=jnp.float32)
                    ctx_parts.append(ctx_u * (1.0 / den))
                ctx = jnp.concatenate(ctx_parts, axis=-1)
                cbuf[rows, :] = ctx.astype(jnp.bfloat16)
                targets = [(1, sem1.at[c]), (3, sem3.at[c])]
                if c == NC - 1:
                    targets.append((2, semd.at[0]))
                for tgt, ssem in targets:
                    r = pltpu.make_async_remote_copy(
                        src_ref=chunk_ref(c), dst_ref=chunk_ref(c),
                        send_sem=ssem, recv_sem=recv_sems.at[c],
                        device_id=(tgt,),
                        device_id_type=pl.DeviceIdType.MESH,
                    )
                    r.start()
                    rdmas.append(r)
            fill_wob()
            odmas = [project_chunk(c) for c in range(NC)]
            for r in rdmas:
                r.wait_send()
            for od in odmas:
                od.wait()

        @pl.when((my == 1) | (my == 3))
        def _():
            fill_wob()
            fwds = []
            odmas = []
            for c in range(NC):
                rc = pltpu.make_async_remote_copy(
                    src_ref=chunk_ref(c), dst_ref=chunk_ref(c),
                    send_sem=sem1.at[c], recv_sem=recv_sems.at[c],
                    device_id=(2,), device_id_type=pl.DeviceIdType.MESH,
                )
                rc.wait_recv()
                if c < NC - 1:
                    fwd_owner = 1 if c % 2 == 0 else 3

                    @pl.when(my == fwd_owner)
                    def _(rc=rc):
                        rc.start()
                    fwds.append((fwd_owner, rc))
                odmas.append(project_chunk(c))
            for owner, r in fwds:
                @pl.when(my == owner)
                def _(r=r):
                    r.wait_send()
            for od in odmas:
                od.wait()

        @pl.when(my == 2)
        def _():
            fill_wob()
            odmas = []
            for c in range(NC):
                rc = pltpu.make_async_remote_copy(
                    src_ref=chunk_ref(c), dst_ref=chunk_ref(c),
                    send_sem=sem1.at[c], recv_sem=recv_sems.at[c],
                    device_id=(0,), device_id_type=pl.DeviceIdType.MESH,
                )
                rc.wait_recv()
                odmas.append(project_chunk(c))
            for od in odmas:
                od.wait()

        @functools.partial(pl.run_scoped,
                           second_barrier=pltpu.SemaphoreType.REGULAR)
        def _(second_barrier):
            for nbr in (left, right):
                pl.semaphore_signal(
                    second_barrier, inc=1,
                    device_id=(nbr,), device_id_type=pl.DeviceIdType.MESH,
                )
            pl.semaphore_wait(second_barrier, 2)

    return pl.pallas_call(
        body,
        out_shape=jax.ShapeDtypeStruct((1, SQ, D), jnp.float32),
        in_specs=[pl.BlockSpec(memory_space=pl.ANY)] * 5,
        out_specs=pl.BlockSpec(memory_space=pl.ANY),
        scratch_shapes=[
            pltpu.VMEM((SQ, D), jnp.float32),
            pltpu.VMEM((D, D), jnp.float32),
            pltpu.VMEM((SKV, HQ, DH), jnp.float32),
            pltpu.VMEM((SKV, HQ, DH), jnp.float32),
            pltpu.VMEM((D, D), jnp.float32),
            pltpu.VMEM((2, SKV, D), jnp.bfloat16),
            pltpu.VMEM((SQ, D), jnp.bfloat16),
            pltpu.VMEM((D, D), jnp.bfloat16),
            pltpu.VMEM((SQ, D), jnp.float32),
            pltpu.SemaphoreType.DMA((3,)),
            pltpu.SemaphoreType.DMA((NC,)),
            pltpu.SemaphoreType.DMA((NC,)),
            pltpu.SemaphoreType.DMA((NC,)),
            pltpu.SemaphoreType.DMA((NC,)),
            pltpu.SemaphoreType.DMA((NC,)),
            pltpu.SemaphoreType.DMA((1,)),
            pltpu.SemaphoreType.DMA((NC,)),
        ],
        compiler_params=pltpu.CompilerParams(
            collective_id=0, vmem_limit_bytes=64 * 1024 * 1024),
    )(x, Wq, K_ext, V_ext, Wo)
